# Initial kernel scaffold; baseline (speedup 1.0000x reference)
import jax
import jax.numpy as jnp
from jax.experimental import pallas as pl


def kernel(x0, x1, x2, x3, x4, x5, x6, x7, x8, x9, x10, x11, x12, x13, x14, x15):
    raise NotImplementedError("write your pallas kernel here")


# speedup vs baseline: 8.0436x; 8.0436x over previous
import functools

import jax
import jax.numpy as jnp
from jax import lax
from jax.experimental import pallas as pl
from jax.experimental.pallas import tpu as pltpu
from jax.experimental.pallas import tpu_sc as plsc

_TOTAL_AMOUNT = 30000000.0
_MIN_AMOUNT = 1500.0
_MIN_RATIO = _MIN_AMOUNT / _TOTAL_AMOUNT
_N = 65536
_NVREG = _N // 16
_NBIN = 1024
_R0 = 32768
_POS_INF_BITS = 0x7F800000


def _body(x0, x1, x2, x3, x4, x5, x6, x7, x8, x9, x10, x11, x12, x13, x14,
          x15, out_ref, inter_ref, xv, hist, rowv, bufv, outv, accr):
  xs = (x0, x1, x2, x3, x4, x5, x6, x7, x8, x9, x10, x11, x12, x13, x14, x15)
  c = lax.axis_index("c")
  s = lax.axis_index("s")
  lane = lax.iota(jnp.int32, 16)
  ones = jnp.ones((16,), jnp.int32)

  @pl.when(c == 0)
  def _core0():
    for a in range(16):
      @pl.when(s == a)
      def _load():
        pltpu.sync_copy(xs[a], xv)

    def zero_hist():
      def zb(i, carry):
        hist[pl.ds(i * 16, 16)] = jnp.zeros((16,), jnp.int32)
        return carry
      lax.fori_loop(0, _NBIN, zb, 0)

    def hist_pass(shift, match_shift, match_prefix):
      zero_hist()

      def pb(i, carry):
        v = xv[pl.ds(i * 16, 16)]
        k = plsc.bitcast(v, jnp.int32)
        b = (k >> shift) & (_NBIN - 1)
        idx = b * 16 + lane
        if match_shift is None:
          plsc.addupdate_scatter(hist, [idx], ones)
        else:
          m = (k >> match_shift) == match_prefix
          plsc.addupdate_scatter(hist, [idx], ones, mask=m)
        return carry
      lax.fori_loop(0, _NVREG, pb, 0)

    def scan_hist(r):
      def gb(g, carry):
        cum, bg, beforeg = carry
        acc = hist[pl.ds(g * 256, 16)]
        for j in range(1, 16):
          acc = acc + hist[pl.ds(g * 256 + j * 16, 16)]
        newcum = cum + jnp.sum(acc)
        crossed = (newcum >= r) & (bg < 0)
        bg = jnp.where(crossed, g, bg)
        beforeg = jnp.where(crossed, cum, beforeg)
        return newcum, bg, beforeg
      _, bg, beforeg = lax.fori_loop(
          0, 64, gb, (jnp.int32(0), jnp.int32(-1), jnp.int32(0)))

      def bb_(j, carry):
        cum, bb, before, at = carry
        sv = jnp.sum(hist[pl.ds((bg * 16 + j) * 16, 16)])
        newcum = cum + sv
        crossed = (newcum >= r) & (bb < 0)
        bb = jnp.where(crossed, bg * 16 + j, bb)
        before = jnp.where(crossed, cum, before)
        at = jnp.where(crossed, newcum, at)
        return newcum, bb, before, at
      _, bb, before, at = lax.fori_loop(
          0, 16, bb_, (beforeg, jnp.int32(-1), jnp.int32(0), jnp.int32(0)))
      return bb, before, at

    hist_pass(20, None, None)
    b1, bef1, _ = scan_hist(_R0)
    hist_pass(10, 20, b1)
    b2, bef2, _ = scan_hist(_R0 - bef1)
    hist_pass(0, 10, (b1 << 10) | b2)
    b3, bef3, at3 = scan_hist(_R0 - bef1 - bef2)
    key0 = (b1 << 20) | (b2 << 10) | b3
    cnt_le = bef1 + bef2 + at3

    accr[...] = jnp.full((16,), 1, jnp.int32) * key0

    @pl.when(cnt_le < _R0 + 1)
    def _next_larger():
      def db(i, acc):
        v = xv[pl.ds(i * 16, 16)]
        k = plsc.bitcast(v, jnp.int32)
        return jnp.minimum(acc, jnp.where(k > key0, k, jnp.int32(_POS_INF_BITS)))
      accr[...] = lax.fori_loop(
          0, _NVREG, db, jnp.full((16,), _POS_INF_BITS, jnp.int32))

    floor_bits = jnp.min(accr[...])
    ceil_v = lax.bitcast_convert_type(key0, jnp.float32)
    floor_v = lax.bitcast_convert_type(floor_bits, jnp.float32)
    median = (ceil_v + floor_v) * 0.5

    rowv[...] = jnp.where(lane == 0, ceil_v,
                          jnp.where(lane == 1, median, 0.0))
    pltpu.sync_copy(rowv, inter_ref.at[s])
    plsc.subcore_barrier()

    @pl.when(s == 0)
    def _combine():
      pltpu.sync_copy(inter_ref, bufv)
      zeros = jnp.zeros((16,), jnp.int32)
      ceils = plsc.load_gather(bufv, [lane, zeros])
      meds = plsc.load_gather(bufv, [lane, zeros + 1])
      scaled = ceils * _MIN_RATIO
      smin = jnp.sum(scaled)
      meets = (meds >= smin).astype(jnp.float32)
      resc = _MIN_AMOUNT * (meds / smin) * meets
      plsc.store_scatter(outv, [lane, zeros],
                         jnp.full((16,), float(_N), jnp.float32))
      plsc.store_scatter(outv, [lane, zeros + 1], meds)
      plsc.store_scatter(outv, [lane, zeros + 2],
                         jnp.ones((16,), jnp.float32))
      plsc.store_scatter(outv, [lane, zeros + 3], resc)
      pltpu.sync_copy(outv, out_ref)


@functools.partial(
    pl.kernel,
    out_type=(jax.ShapeDtypeStruct((16, 4), jnp.float32),
              jax.ShapeDtypeStruct((16, 16), jnp.float32)),
    mesh=plsc.VectorSubcoreMesh(core_axis_name="c", subcore_axis_name="s"),
    compiler_params=pltpu.CompilerParams(needs_layout_passes=False),
    scratch_types=[
        pltpu.VMEM((_N,), jnp.float32),
        pltpu.VMEM((_NBIN * 16,), jnp.int32),
        pltpu.VMEM((16,), jnp.float32),
        pltpu.VMEM((16, 16), jnp.float32),
        pltpu.VMEM((16, 4), jnp.float32),
        pltpu.VMEM((16,), jnp.int32),
    ],
)
def _allocator(*refs):
  _body(*refs)


def kernel(x0, x1, x2, x3, x4, x5, x6, x7, x8, x9, x10, x11, x12, x13, x14,
           x15):
  out, _ = _allocator(x0, x1, x2, x3, x4, x5, x6, x7, x8, x9, x10, x11, x12,
                      x13, x14, x15)
  return out


# speedup vs baseline: 10.4532x; 1.2996x over previous
import functools

import jax
import jax.numpy as jnp
from jax import lax
from jax.experimental import pallas as pl
from jax.experimental.pallas import tpu as pltpu
from jax.experimental.pallas import tpu_sc as plsc

_TOTAL_AMOUNT = 30000000.0
_MIN_AMOUNT = 1500.0
_MIN_RATIO = _MIN_AMOUNT / _TOTAL_AMOUNT
_N = 65536
_NVREG = _N // 16
_NBIN = 1024
_R0 = 32768
_POS_INF_BITS = 0x7F800000


def _body(x0, x1, x2, x3, x4, x5, x6, x7, x8, x9, x10, x11, x12, x13, x14,
          x15, out_ref, inter_ref, xv, hist, rowv, bufv, outv, accr):
  xs = (x0, x1, x2, x3, x4, x5, x6, x7, x8, x9, x10, x11, x12, x13, x14, x15)
  c = lax.axis_index("c")
  s = lax.axis_index("s")
  lane = lax.iota(jnp.int32, 16)
  ones = jnp.ones((16,), jnp.int32)

  @pl.when(c == 0)
  def _core0():
    for a in range(16):
      @pl.when(s == a)
      def _load():
        pltpu.sync_copy(xs[a], xv)

    _U = 8

    def zero_hist():
      zz = jnp.zeros((16,), jnp.int32)

      def zb(i, carry):
        for u in range(_U):
          hist[pl.ds(i * (16 * _U) + u * 16, 16)] = zz
        return carry
      lax.fori_loop(0, _NBIN // _U, zb, 0)

    def hist_pass(shift, match_shift, match_prefix):
      zero_hist()

      def pb(i, carry):
        for u in range(_U):
          v = xv[pl.ds(i * (16 * _U) + u * 16, 16)]
          k = plsc.bitcast(v, jnp.int32)
          b = (k >> shift) & (_NBIN - 1)
          idx = b * 16 + lane
          if match_shift is None:
            plsc.addupdate_scatter(hist, [idx], ones)
          else:
            m = (k >> match_shift) == match_prefix
            plsc.addupdate_scatter(hist, [idx], ones, mask=m)
        return carry
      lax.fori_loop(0, _NVREG // _U, pb, 0)

    def scan_hist(r):
      def gb(g, carry):
        cum, bg, beforeg = carry
        acc = hist[pl.ds(g * 256, 16)]
        for j in range(1, 16):
          acc = acc + hist[pl.ds(g * 256 + j * 16, 16)]
        newcum = cum + jnp.sum(acc)
        crossed = (newcum >= r) & (bg < 0)
        bg = jnp.where(crossed, g, bg)
        beforeg = jnp.where(crossed, cum, beforeg)
        return newcum, bg, beforeg
      _, bg, beforeg = lax.fori_loop(
          0, 64, gb, (jnp.int32(0), jnp.int32(-1), jnp.int32(0)))

      def bb_(j, carry):
        cum, bb, before, at = carry
        sv = jnp.sum(hist[pl.ds((bg * 16 + j) * 16, 16)])
        newcum = cum + sv
        crossed = (newcum >= r) & (bb < 0)
        bb = jnp.where(crossed, bg * 16 + j, bb)
        before = jnp.where(crossed, cum, before)
        at = jnp.where(crossed, newcum, at)
        return newcum, bb, before, at
      _, bb, before, at = lax.fori_loop(
          0, 16, bb_, (beforeg, jnp.int32(-1), jnp.int32(0), jnp.int32(0)))
      return bb, before, at

    hist_pass(20, None, None)
    b1, bef1, _ = scan_hist(_R0)
    hist_pass(10, 20, b1)
    b2, bef2, _ = scan_hist(_R0 - bef1)
    hist_pass(0, 10, (b1 << 10) | b2)
    b3, bef3, at3 = scan_hist(_R0 - bef1 - bef2)
    key0 = (b1 << 20) | (b2 << 10) | b3
    cnt_le = bef1 + bef2 + at3

    accr[...] = jnp.full((16,), 1, jnp.int32) * key0

    @pl.when(cnt_le < _R0 + 1)
    def _next_larger():
      inf = jnp.full((16,), _POS_INF_BITS, jnp.int32)

      def db(i, accs):
        accs = list(accs)
        for u in range(_U):
          v = xv[pl.ds(i * (16 * _U) + u * 16, 16)]
          k = plsc.bitcast(v, jnp.int32)
          accs[u % 4] = jnp.minimum(
              accs[u % 4], jnp.where(k > key0, k, jnp.int32(_POS_INF_BITS)))
        return tuple(accs)
      a0, a1, a2, a3 = lax.fori_loop(
          0, _NVREG // _U, db, (inf, inf, inf, inf))
      accr[...] = jnp.minimum(jnp.minimum(a0, a1), jnp.minimum(a2, a3))

    floor_bits = jnp.min(accr[...])
    ceil_v = lax.bitcast_convert_type(key0, jnp.float32)
    floor_v = lax.bitcast_convert_type(floor_bits, jnp.float32)
    median = (ceil_v + floor_v) * 0.5

    rowv[...] = jnp.where(lane == 0, ceil_v,
                          jnp.where(lane == 1, median, 0.0))
    pltpu.sync_copy(rowv, inter_ref.at[s])
    plsc.subcore_barrier()

    @pl.when(s == 0)
    def _combine():
      pltpu.sync_copy(inter_ref, bufv)
      zeros = jnp.zeros((16,), jnp.int32)
      ceils = plsc.load_gather(bufv, [lane, zeros])
      meds = plsc.load_gather(bufv, [lane, zeros + 1])
      scaled = ceils * _MIN_RATIO
      smin = jnp.sum(scaled)
      meets = (meds >= smin).astype(jnp.float32)
      resc = _MIN_AMOUNT * (meds / smin) * meets
      plsc.store_scatter(outv, [lane, zeros],
                         jnp.full((16,), float(_N), jnp.float32))
      plsc.store_scatter(outv, [lane, zeros + 1], meds)
      plsc.store_scatter(outv, [lane, zeros + 2],
                         jnp.ones((16,), jnp.float32))
      plsc.store_scatter(outv, [lane, zeros + 3], resc)
      pltpu.sync_copy(outv, out_ref)


@functools.partial(
    pl.kernel,
    out_type=(jax.ShapeDtypeStruct((16, 4), jnp.float32),
              jax.ShapeDtypeStruct((16, 16), jnp.float32)),
    mesh=plsc.VectorSubcoreMesh(core_axis_name="c", subcore_axis_name="s"),
    compiler_params=pltpu.CompilerParams(needs_layout_passes=False),
    scratch_types=[
        pltpu.VMEM((_N,), jnp.float32),
        pltpu.VMEM((_NBIN * 16,), jnp.int32),
        pltpu.VMEM((16,), jnp.float32),
        pltpu.VMEM((16, 16), jnp.float32),
        pltpu.VMEM((16, 4), jnp.float32),
        pltpu.VMEM((16,), jnp.int32),
    ],
)
def _allocator(*refs):
  _body(*refs)


def kernel(x0, x1, x2, x3, x4, x5, x6, x7, x8, x9, x10, x11, x12, x13, x14,
           x15):
  out, _ = _allocator(x0, x1, x2, x3, x4, x5, x6, x7, x8, x9, x10, x11, x12,
                      x13, x14, x15)
  return out


# speedup vs baseline: 10.4651x; 1.0011x over previous
import functools

import jax
import jax.numpy as jnp
from jax import lax
from jax.experimental import pallas as pl
from jax.experimental.pallas import tpu as pltpu
from jax.experimental.pallas import tpu_sc as plsc

_TOTAL_AMOUNT = 30000000.0
_MIN_AMOUNT = 1500.0
_MIN_RATIO = _MIN_AMOUNT / _TOTAL_AMOUNT
_N = 65536
_NVREG = _N // 16
_NBIN = 1024
_R0 = 32768
_POS_INF_BITS = 0x7F800000


def _body(x0, x1, x2, x3, x4, x5, x6, x7, x8, x9, x10, x11, x12, x13, x14,
          x15, out_ref, inter_ref, xv, hist, rowv, bufv, outv, accr):
  xs = (x0, x1, x2, x3, x4, x5, x6, x7, x8, x9, x10, x11, x12, x13, x14, x15)
  c = lax.axis_index("c")
  s = lax.axis_index("s")
  lane = lax.iota(jnp.int32, 16)
  ones = jnp.ones((16,), jnp.int32)

  @pl.when(c == 0)
  def _core0():
    with jax.named_scope("dma_in"):
      for a in range(16):
        @pl.when(s == a)
        def _load():
          pltpu.sync_copy(xs[a], xv)

    _U = 8

    def zero_hist():
      zz = jnp.zeros((16,), jnp.int32)

      def zb(i, carry):
        for u in range(_U):
          hist[pl.ds(i * (16 * _U) + u * 16, 16)] = zz
        return carry
      lax.fori_loop(0, _NBIN // _U, zb, 0)

    def hist_pass(shift, match_shift, match_prefix):
      zero_hist()

      def pb(i, carry):
        for u in range(_U):
          v = xv[pl.ds(i * (16 * _U) + u * 16, 16)]
          k = plsc.bitcast(v, jnp.int32)
          b = (k >> shift) & (_NBIN - 1)
          idx = b * 16 + lane
          if match_shift is None:
            plsc.addupdate_scatter(hist, [idx], ones)
          else:
            m = (k >> match_shift) == match_prefix
            plsc.addupdate_scatter(hist, [idx], ones, mask=m)
        return carry
      lax.fori_loop(0, _NVREG // _U, pb, 0)

    def scan_hist(r):
      def gb(g, carry):
        cum, bg, beforeg = carry
        acc = hist[pl.ds(g * 256, 16)]
        for j in range(1, 16):
          acc = acc + hist[pl.ds(g * 256 + j * 16, 16)]
        newcum = cum + jnp.sum(acc)
        crossed = (newcum >= r) & (bg < 0)
        bg = jnp.where(crossed, g, bg)
        beforeg = jnp.where(crossed, cum, beforeg)
        return newcum, bg, beforeg
      _, bg, beforeg = lax.fori_loop(
          0, 64, gb, (jnp.int32(0), jnp.int32(-1), jnp.int32(0)))

      def bb_(j, carry):
        cum, bb, before, at = carry
        sv = jnp.sum(hist[pl.ds((bg * 16 + j) * 16, 16)])
        newcum = cum + sv
        crossed = (newcum >= r) & (bb < 0)
        bb = jnp.where(crossed, bg * 16 + j, bb)
        before = jnp.where(crossed, cum, before)
        at = jnp.where(crossed, newcum, at)
        return newcum, bb, before, at
      _, bb, before, at = lax.fori_loop(
          0, 16, bb_, (beforeg, jnp.int32(-1), jnp.int32(0), jnp.int32(0)))
      return bb, before, at

    with jax.named_scope("pass1"):
      hist_pass(20, None, None)
    with jax.named_scope("scan1"):
      b1, bef1, _ = scan_hist(_R0)
    with jax.named_scope("pass2"):
      hist_pass(10, 20, b1)
    with jax.named_scope("scan2"):
      b2, bef2, _ = scan_hist(_R0 - bef1)
    with jax.named_scope("pass3"):
      hist_pass(0, 10, (b1 << 10) | b2)
    with jax.named_scope("scan3"):
      b3, bef3, at3 = scan_hist(_R0 - bef1 - bef2)
    key0 = (b1 << 20) | (b2 << 10) | b3
    cnt_le = bef1 + bef2 + at3

    accr[...] = jnp.full((16,), 1, jnp.int32) * key0

    @pl.when(cnt_le < _R0 + 1)
    def _next_larger():
      inf = jnp.full((16,), _POS_INF_BITS, jnp.int32)

      def db(i, accs):
        accs = list(accs)
        for u in range(_U):
          v = xv[pl.ds(i * (16 * _U) + u * 16, 16)]
          k = plsc.bitcast(v, jnp.int32)
          accs[u % 4] = jnp.minimum(
              accs[u % 4], jnp.where(k > key0, k, jnp.int32(_POS_INF_BITS)))
        return tuple(accs)
      a0, a1, a2, a3 = lax.fori_loop(
          0, _NVREG // _U, db, (inf, inf, inf, inf))
      accr[...] = jnp.minimum(jnp.minimum(a0, a1), jnp.minimum(a2, a3))

    floor_bits = jnp.min(accr[...])
    ceil_v = lax.bitcast_convert_type(key0, jnp.float32)
    floor_v = lax.bitcast_convert_type(floor_bits, jnp.float32)
    median = (ceil_v + floor_v) * 0.5

    rowv[...] = jnp.where(lane == 0, ceil_v,
                          jnp.where(lane == 1, median, 0.0))
    pltpu.sync_copy(rowv, inter_ref.at[s])
    plsc.subcore_barrier()

    @pl.when(s == 0)
    def _combine():
      pltpu.sync_copy(inter_ref, bufv)
      zeros = jnp.zeros((16,), jnp.int32)
      ceils = plsc.load_gather(bufv, [lane, zeros])
      meds = plsc.load_gather(bufv, [lane, zeros + 1])
      scaled = ceils * _MIN_RATIO
      smin = jnp.sum(scaled)
      meets = (meds >= smin).astype(jnp.float32)
      resc = _MIN_AMOUNT * (meds / smin) * meets
      plsc.store_scatter(outv, [lane, zeros],
                         jnp.full((16,), float(_N), jnp.float32))
      plsc.store_scatter(outv, [lane, zeros + 1], meds)
      plsc.store_scatter(outv, [lane, zeros + 2],
                         jnp.ones((16,), jnp.float32))
      plsc.store_scatter(outv, [lane, zeros + 3], resc)
      pltpu.sync_copy(outv, out_ref)


@functools.partial(
    pl.kernel,
    out_type=(jax.ShapeDtypeStruct((16, 4), jnp.float32),
              jax.ShapeDtypeStruct((16, 16), jnp.float32)),
    mesh=plsc.VectorSubcoreMesh(core_axis_name="c", subcore_axis_name="s"),
    compiler_params=pltpu.CompilerParams(needs_layout_passes=False),
    scratch_types=[
        pltpu.VMEM((_N,), jnp.float32),
        pltpu.VMEM((_NBIN * 16,), jnp.int32),
        pltpu.VMEM((16,), jnp.float32),
        pltpu.VMEM((16, 16), jnp.float32),
        pltpu.VMEM((16, 4), jnp.float32),
        pltpu.VMEM((16,), jnp.int32),
    ],
)
def _allocator(*refs):
  _body(*refs)


def kernel(x0, x1, x2, x3, x4, x5, x6, x7, x8, x9, x10, x11, x12, x13, x14,
           x15):
  out, _ = _allocator(x0, x1, x2, x3, x4, x5, x6, x7, x8, x9, x10, x11, x12,
                      x13, x14, x15)
  return out


# speedup vs baseline: 27.7079x; 2.6477x over previous
import functools

import jax
import jax.numpy as jnp
from jax import lax
from jax.experimental import pallas as pl
from jax.experimental.pallas import tpu as pltpu
from jax.experimental.pallas import tpu_sc as plsc

_TOTAL_AMOUNT = 30000000.0
_MIN_AMOUNT = 1500.0
_MIN_RATIO = _MIN_AMOUNT / _TOTAL_AMOUNT
_N = 65536
_NVREG = _N // 16
_NBIN = 1024
_R0 = 32768
_POS_INF_BITS = 0x7F800000


def _body(x0, x1, x2, x3, x4, x5, x6, x7, x8, x9, x10, x11, x12, x13, x14,
          x15, out_ref, inter_ref, xv, hist, rowv, bufv, outv, accr):
  xs = (x0, x1, x2, x3, x4, x5, x6, x7, x8, x9, x10, x11, x12, x13, x14, x15)
  c = lax.axis_index("c")
  s = lax.axis_index("s")
  lane = lax.iota(jnp.int32, 16)
  ones = jnp.ones((16,), jnp.int32)

  @pl.when(c == 0)
  def _core0():
    with jax.named_scope("dma_in"):
      for a in range(16):
        @pl.when(s == a)
        def _load():
          pltpu.sync_copy(xs[a], xv)

    _U = 8

    def zero_hist():
      zz = jnp.zeros((16,), jnp.int32)

      @plsc.parallel_loop(0, _NBIN * 16, step=16, unroll=_U)
      def _zb(i):
        hist[pl.ds(i, 16)] = zz

    def hist_pass(shift, match_shift, match_prefix):
      zero_hist()

      @plsc.parallel_loop(0, _N, step=16, unroll=_U)
      def _pb(i):
        v = xv[pl.ds(i, 16)]
        k = plsc.bitcast(v, jnp.int32)
        b = (k >> shift) & (_NBIN - 1)
        idx = b * 16 + lane
        if match_shift is None:
          plsc.addupdate_scatter(hist, [idx], ones)
        else:
          m = (k >> match_shift) == match_prefix
          plsc.addupdate_scatter(hist, [idx], ones, mask=m)

    def scan_hist(r):
      def gb(g, carry):
        cum, bg, beforeg = carry
        acc = hist[pl.ds(g * 256, 16)]
        for j in range(1, 16):
          acc = acc + hist[pl.ds(g * 256 + j * 16, 16)]
        newcum = cum + jnp.sum(acc)
        crossed = (newcum >= r) & (bg < 0)
        bg = jnp.where(crossed, g, bg)
        beforeg = jnp.where(crossed, cum, beforeg)
        return newcum, bg, beforeg
      _, bg, beforeg = lax.fori_loop(
          0, 64, gb, (jnp.int32(0), jnp.int32(-1), jnp.int32(0)))

      def bb_(j, carry):
        cum, bb, before, at = carry
        sv = jnp.sum(hist[pl.ds((bg * 16 + j) * 16, 16)])
        newcum = cum + sv
        crossed = (newcum >= r) & (bb < 0)
        bb = jnp.where(crossed, bg * 16 + j, bb)
        before = jnp.where(crossed, cum, before)
        at = jnp.where(crossed, newcum, at)
        return newcum, bb, before, at
      _, bb, before, at = lax.fori_loop(
          0, 16, bb_, (beforeg, jnp.int32(-1), jnp.int32(0), jnp.int32(0)))
      return bb, before, at

    with jax.named_scope("pass1"):
      hist_pass(20, None, None)
    with jax.named_scope("scan1"):
      b1, bef1, _ = scan_hist(_R0)
    with jax.named_scope("pass2"):
      hist_pass(10, 20, b1)
    with jax.named_scope("scan2"):
      b2, bef2, _ = scan_hist(_R0 - bef1)
    with jax.named_scope("pass3"):
      hist_pass(0, 10, (b1 << 10) | b2)
    with jax.named_scope("scan3"):
      b3, bef3, at3 = scan_hist(_R0 - bef1 - bef2)
    key0 = (b1 << 20) | (b2 << 10) | b3
    cnt_le = bef1 + bef2 + at3

    accr[...] = jnp.full((16,), 1, jnp.int32) * key0

    @pl.when(cnt_le < _R0 + 1)
    def _next_larger():
      inf = jnp.full((16,), _POS_INF_BITS, jnp.int32)

      @plsc.parallel_loop(0, _N, step=64, unroll=2, carry=(inf, inf, inf, inf))
      def db(i, accs):
        accs = list(accs)
        for u in range(4):
          v = xv[pl.ds(i + u * 16, 16)]
          k = plsc.bitcast(v, jnp.int32)
          accs[u] = jnp.minimum(
              accs[u], jnp.where(k > key0, k, jnp.int32(_POS_INF_BITS)))
        return tuple(accs)
      a0, a1, a2, a3 = db
      accr[...] = jnp.minimum(jnp.minimum(a0, a1), jnp.minimum(a2, a3))

    floor_bits = jnp.min(accr[...])
    ceil_v = lax.bitcast_convert_type(key0, jnp.float32)
    floor_v = lax.bitcast_convert_type(floor_bits, jnp.float32)
    median = (ceil_v + floor_v) * 0.5

    rowv[...] = jnp.where(lane == 0, ceil_v,
                          jnp.where(lane == 1, median, 0.0))
    pltpu.sync_copy(rowv, inter_ref.at[s])
    plsc.subcore_barrier()

    @pl.when(s == 0)
    def _combine():
      pltpu.sync_copy(inter_ref, bufv)
      zeros = jnp.zeros((16,), jnp.int32)
      ceils = plsc.load_gather(bufv, [lane, zeros])
      meds = plsc.load_gather(bufv, [lane, zeros + 1])
      scaled = ceils * _MIN_RATIO
      smin = jnp.sum(scaled)
      meets = (meds >= smin).astype(jnp.float32)
      resc = _MIN_AMOUNT * (meds / smin) * meets
      plsc.store_scatter(outv, [lane, zeros],
                         jnp.full((16,), float(_N), jnp.float32))
      plsc.store_scatter(outv, [lane, zeros + 1], meds)
      plsc.store_scatter(outv, [lane, zeros + 2],
                         jnp.ones((16,), jnp.float32))
      plsc.store_scatter(outv, [lane, zeros + 3], resc)
      pltpu.sync_copy(outv, out_ref)


@functools.partial(
    pl.kernel,
    out_type=(jax.ShapeDtypeStruct((16, 4), jnp.float32),
              jax.ShapeDtypeStruct((16, 16), jnp.float32)),
    mesh=plsc.VectorSubcoreMesh(core_axis_name="c", subcore_axis_name="s"),
    compiler_params=pltpu.CompilerParams(needs_layout_passes=False),
    scratch_types=[
        pltpu.VMEM((_N,), jnp.float32),
        pltpu.VMEM((_NBIN * 16,), jnp.int32),
        pltpu.VMEM((16,), jnp.float32),
        pltpu.VMEM((16, 16), jnp.float32),
        pltpu.VMEM((16, 4), jnp.float32),
        pltpu.VMEM((16,), jnp.int32),
    ],
)
def _allocator(*refs):
  _body(*refs)


def kernel(x0, x1, x2, x3, x4, x5, x6, x7, x8, x9, x10, x11, x12, x13, x14,
           x15):
  out, _ = _allocator(x0, x1, x2, x3, x4, x5, x6, x7, x8, x9, x10, x11, x12,
                      x13, x14, x15)
  return out
